# Initial kernel scaffold; baseline (speedup 1.0000x reference)
import jax
import jax.numpy as jnp
from jax.experimental import pallas as pl


def kernel(x, edge_index, edge_weight, attention, Wz, bz, Wr, br, Wh, bh, Lzw, Lzb, Lrw, Lrb, Lhw, Lhb, lin_w, lin_b):
    raise NotImplementedError("write your pallas kernel here")


# speedup vs baseline: 241.3997x; 241.3997x over previous
import functools

import jax
import jax.numpy as jnp
from jax import lax
from jax.experimental import pallas as pl
from jax.experimental.pallas import tpu as pltpu
from jax.experimental.pallas import tpu_sc as plsc

NC = 2
NS = 16
NW = NC * NS
LANES = 128


def _deg_body(col_hbm, ew_hbm, init_hbm, degp_hbm, deg_sh, colv, wv,
              *, npad, rpw, nchunk, ck, extra):
    cid = lax.axis_index("c")
    sid = lax.axis_index("s")
    wid = sid * NC + cid
    nps = npad // NS
    nsl = pl.ds(sid * nps, nps)
    pltpu.sync_copy(init_hbm.at[cid, nsl], deg_sh.at[nsl])
    plsc.subcore_barrier()

    def chunk(ci, carry):
        r0 = wid * rpw + ci * ck
        pltpu.sync_copy(col_hbm.at[pl.ds(r0, ck)], colv)
        pltpu.sync_copy(ew_hbm.at[pl.ds(r0 * LANES, ck * LANES)], wv)
        for j in range(ck):
            pltpu.sync_copy(wv.at[pl.ds(j * LANES, LANES)],
                            deg_sh.at[colv.at[j]], add=True)
        return carry

    lax.fori_loop(0, nchunk, chunk, 0)

    @pl.when(wid < extra)
    def _():
        r0 = rpw * NW + wid
        pltpu.sync_copy(col_hbm.at[pl.ds(r0, 1)], colv.at[pl.ds(0, 1)])
        pltpu.sync_copy(ew_hbm.at[pl.ds(r0 * LANES, LANES)],
                        wv.at[pl.ds(0, LANES)])
        pltpu.sync_copy(wv.at[pl.ds(0, LANES)], deg_sh.at[colv.at[0]],
                        add=True)

    plsc.subcore_barrier()
    pltpu.sync_copy(deg_sh.at[nsl], degp_hbm.at[cid, nsl])


def _agg_body(row_hbm, col_hbm, ew_hbm, y_hbm, zero_hbm, aggp_hbm,
              agg_sh, rowv, colv, wv, xg, ctr, sem,
              *, npad, p, rpw, nchunk, ck, extra):
    cid = lax.axis_index("c")
    sid = lax.axis_index("s")
    wid = sid * NC + cid
    nps = npad // NS
    nsl = pl.ds(sid * nps, nps)
    pltpu.sync_copy(zero_hbm.at[nsl], agg_sh.at[nsl])
    plsc.subcore_barrier()

    def process(r0, nrows):
        pltpu.sync_copy(row_hbm.at[pl.ds(r0, nrows)],
                        rowv.at[pl.ds(0, nrows)])
        pltpu.sync_copy(col_hbm.at[pl.ds(r0, nrows)],
                        colv.at[pl.ds(0, nrows)])
        pltpu.sync_copy(ew_hbm.at[pl.ds(r0 * LANES, nrows * LANES)],
                        wv.at[pl.ds(0, nrows * LANES)])
        for j in range(nrows):
            pltpu.async_copy(y_hbm.at[rowv.at[j]],
                             xg.at[pl.ds(j * LANES, LANES)], sem).wait()

        def egrp(k, carry):
            lin = k * 16 + lax.iota(jnp.int32, 16)
            e = lin >> 3
            t = lin & 7
            s16 = plsc.load_gather(wv, [e])
            xv = plsc.load_gather(xg, [e, t])
            plsc.store_scatter(ctr, [e, t], s16 * xv)
            return carry

        lax.fori_loop(0, nrows * LANES * p // 16, egrp, 0)

        for j in range(nrows):
            pltpu.sync_copy(ctr.at[pl.ds(j * LANES, LANES)],
                            agg_sh.at[colv.at[j]], add=True)

    def chunk(ci, carry):
        process(wid * rpw + ci * ck, ck)
        return carry

    lax.fori_loop(0, nchunk, chunk, 0)

    @pl.when(wid < extra)
    def _():
        process(rpw * NW + wid, 1)

    plsc.subcore_barrier()
    pltpu.sync_copy(agg_sh.at[nsl], aggp_hbm.at[cid, nsl])


def _dinv_body(degp_ref, xt3_ref, dinv_ref, yt3_ref):
    s = degp_ref[0] + degp_ref[1]
    r0 = lax.rsqrt(s)
    di = r0 * (1.5 - 0.5 * s * r0 * r0)
    dinv_ref[...] = di
    xb = xt3_ref[...].astype(jnp.bfloat16).astype(jnp.float32)
    yt3_ref[...] = xb * di[None, :, :]


def _dense_body(aggp_ref, x_ref, dinv_ref, s_ref, g2_ref, wz_ref, bz_ref,
                wh_ref, bh_ref, lz_ref, lh_ref, lzb_ref, lhb_ref, pr_ref,
                lw_ref, lb_ref, out_ref):
    hi = lax.Precision.HIGHEST
    di = dinv_ref[...]
    xb = x_ref[...].astype(jnp.bfloat16).astype(jnp.float32)
    a = (aggp_ref[0] + aggp_ref[1] + di * xb) * di
    a256 = jnp.dot(a, s_ref[...], preferred_element_type=jnp.float32,
                   precision=hi)
    cz = (a256 * wz_ref[...] + bz_ref[...]).astype(jnp.bfloat16)
    ch_ = (a256 * wh_ref[...] + bh_ref[...]).astype(jnp.bfloat16)
    u = jnp.dot(cz, lz_ref[...], preferred_element_type=jnp.float32) + lzb_ref[...]
    v = jnp.dot(ch_, lh_ref[...], preferred_element_type=jnp.float32) + lhb_ref[...]
    w = (1.0 - jax.nn.sigmoid(u)) * jnp.tanh(v) * pr_ref[...]
    acc = jnp.dot(w, g2_ref[...], preferred_element_type=jnp.float32,
                  precision=hi)
    h = jnp.maximum(acc, 0.0).astype(jnp.bfloat16)
    out_ref[...] = (jnp.dot(h, lw_ref[...], preferred_element_type=jnp.float32)
                    + lb_ref[...])


def kernel(x, edge_index, edge_weight, attention, Wz, bz, Wr, br, Wh, bh,
           Lzw, Lzb, Lrw, Lrb, Lhw, Lhb, lin_w, lin_b):
    n, p = x.shape
    e = edge_index.shape[1]
    hid = Wz.shape[1]
    assert e % LANES == 0
    erows = e // LANES
    rpw = erows // NW
    extra = erows - rpw * NW
    ck = 1
    for cand in range(15, 0, -1):
        if rpw % cand == 0:
            ck = cand
            break
    nchunk = rpw // ck
    align = NS * LANES
    npad = ((n + align - 1) // align) * align

    row2 = edge_index[0].reshape(erows, LANES)
    col2 = edge_index[1].reshape(erows, LANES)
    xp = jnp.concatenate([x, jnp.zeros((npad - n, p), jnp.float32)], axis=0)
    init_deg = jnp.concatenate(
        [jnp.ones((1, npad), jnp.float32), jnp.zeros((1, npad), jnp.float32)],
        axis=0)
    zero8 = jnp.zeros((npad, p), jnp.float32)

    mesh = plsc.VectorSubcoreMesh(core_axis_name="c", subcore_axis_name="s")
    sc_params = pltpu.CompilerParams(use_tc_tiling_on_sc=False,
                                     needs_layout_passes=False)

    deg_fn = pl.kernel(
        functools.partial(_deg_body, npad=npad, rpw=rpw, nchunk=nchunk,
                          ck=ck, extra=extra),
        out_type=jax.ShapeDtypeStruct((NC, npad), jnp.float32),
        mesh=mesh,
        scratch_types=[
            pltpu.VMEM_SHARED((npad,), jnp.float32),
            pltpu.VMEM((ck, LANES), jnp.int32),
            pltpu.VMEM((ck * LANES,), jnp.float32),
        ],
        compiler_params=sc_params,
    )
    degp = deg_fn(col2, edge_weight, init_deg)

    dinv2, yt3 = pl.pallas_call(
        _dinv_body,
        out_shape=(
            jax.ShapeDtypeStruct((npad // LANES, LANES), jnp.float32),
            jax.ShapeDtypeStruct((p, npad // LANES, LANES), jnp.float32),
        ),
    )(degp.reshape(NC, npad // LANES, LANES),
      xp.T.reshape(p, npad // LANES, LANES))
    y = yt3.reshape(p, npad).T

    agg_fn = pl.kernel(
        functools.partial(_agg_body, npad=npad, p=p, rpw=rpw, nchunk=nchunk,
                          ck=ck, extra=extra),
        out_type=jax.ShapeDtypeStruct((NC, npad, p), jnp.float32),
        mesh=mesh,
        scratch_types=[
            pltpu.VMEM_SHARED((npad, p), jnp.float32),
            pltpu.VMEM((ck, LANES), jnp.int32),
            pltpu.VMEM((ck, LANES), jnp.int32),
            pltpu.VMEM((ck * LANES,), jnp.float32),
            pltpu.VMEM((ck * LANES, p), jnp.float32),
            pltpu.VMEM((ck * LANES, p), jnp.float32),
            pltpu.SemaphoreType.DMA,
        ],
        compiler_params=sc_params,
    )
    aggp = agg_fn(row2, col2, edge_weight, y, zero8)

    probs = jax.nn.softmax(attention)
    bf = jnp.bfloat16
    wz256 = jnp.tile(Wz[0].astype(bf).astype(jnp.float32), p)[None, :]
    bz256 = jnp.tile(bz, p)[None, :]
    wh256 = jnp.tile(Wh[0].astype(bf).astype(jnp.float32), p)[None, :]
    bh256 = jnp.tile(bh, p)[None, :]
    eye_p = jnp.eye(p, dtype=jnp.float32)
    lzblk = jnp.kron(eye_p, Lzw[:hid]).astype(bf)
    lhblk = jnp.kron(eye_p, Lhw[:hid]).astype(bf)
    lzb256 = jnp.tile(Lzb, p)[None, :]
    lhb256 = jnp.tile(Lhb, p)[None, :]
    pr = jnp.repeat(probs, hid)[None, :]
    sel = jnp.kron(eye_p, jnp.ones((1, hid), jnp.float32))
    g2 = jnp.kron(jnp.ones((p, 1), jnp.float32), jnp.eye(hid, dtype=jnp.float32))
    ph = p * hid

    br_rows = npad // 16
    grid = 16
    out_pad = pl.pallas_call(
        _dense_body,
        grid=(grid,),
        in_specs=[
            pl.BlockSpec((NC, br_rows, p), lambda i: (0, i, 0)),
            pl.BlockSpec((br_rows, p), lambda i: (i, 0)),
            pl.BlockSpec((br_rows, 1), lambda i: (i, 0)),
            pl.BlockSpec((p, ph), lambda i: (0, 0)),
            pl.BlockSpec((ph, hid), lambda i: (0, 0)),
            pl.BlockSpec((1, ph), lambda i: (0, 0)),
            pl.BlockSpec((1, ph), lambda i: (0, 0)),
            pl.BlockSpec((1, ph), lambda i: (0, 0)),
            pl.BlockSpec((1, ph), lambda i: (0, 0)),
            pl.BlockSpec((ph, ph), lambda i: (0, 0)),
            pl.BlockSpec((ph, ph), lambda i: (0, 0)),
            pl.BlockSpec((1, ph), lambda i: (0, 0)),
            pl.BlockSpec((1, ph), lambda i: (0, 0)),
            pl.BlockSpec((1, ph), lambda i: (0, 0)),
            pl.BlockSpec((hid, 1), lambda i: (0, 0)),
            pl.BlockSpec((1, 1), lambda i: (0, 0)),
        ],
        out_specs=pl.BlockSpec((br_rows, 1), lambda i: (i, 0)),
        out_shape=jax.ShapeDtypeStruct((npad, 1), jnp.float32),
    )(aggp, xp, dinv2.reshape(npad, 1), sel, g2, wz256, bz256, wh256, bh256,
      lzblk, lhblk, lzb256, lhb256, pr, lin_w.astype(bf), lin_b.reshape(1, 1))

    return out_pad[:n]


# speedup vs baseline: 259.1590x; 1.0736x over previous
import functools

import jax
import jax.numpy as jnp
from jax import lax
from jax.experimental import pallas as pl
from jax.experimental.pallas import tpu as pltpu
from jax.experimental.pallas import tpu_sc as plsc

NC = 2
NS = 16
NW = NC * NS
LANES = 128


def _deg_body(col_hbm, ew_hbm, init_hbm, degp_hbm, deg_sh, colv, wv,
              *, npad, rpw, nchunk, ck, extra):
    cid = lax.axis_index("c")
    sid = lax.axis_index("s")
    wid = sid * NC + cid
    nps = npad // NS
    nsl = pl.ds(sid * nps, nps)
    pltpu.sync_copy(init_hbm.at[cid, nsl], deg_sh.at[nsl])
    plsc.subcore_barrier()

    def chunk(ci, carry):
        r0 = wid * rpw + ci * ck
        pltpu.sync_copy(col_hbm.at[pl.ds(r0, ck)], colv)
        pltpu.sync_copy(ew_hbm.at[pl.ds(r0 * LANES, ck * LANES)], wv)
        for j in range(ck):
            pltpu.sync_copy(wv.at[pl.ds(j * LANES, LANES)],
                            deg_sh.at[colv.at[j]], add=True)
        return carry

    lax.fori_loop(0, nchunk, chunk, 0)

    @pl.when(wid < extra)
    def _():
        r0 = rpw * NW + wid
        pltpu.sync_copy(col_hbm.at[pl.ds(r0, 1)], colv.at[pl.ds(0, 1)])
        pltpu.sync_copy(ew_hbm.at[pl.ds(r0 * LANES, LANES)],
                        wv.at[pl.ds(0, LANES)])
        pltpu.sync_copy(wv.at[pl.ds(0, LANES)], deg_sh.at[colv.at[0]],
                        add=True)

    plsc.subcore_barrier()
    pltpu.sync_copy(deg_sh.at[nsl], degp_hbm.at[cid, nsl])


def _agg_body(row_hbm, col_hbm, ew_hbm, y_hbm, zero_hbm, aggp_hbm,
              agg_sh, rowv, colv, wv, xg, ctr, sem,
              *, npad, p, rpw, nchunk, ck, extra):
    cid = lax.axis_index("c")
    sid = lax.axis_index("s")
    wid = sid * NC + cid
    nps = npad // NS
    nsl = pl.ds(sid * nps, nps)
    pltpu.sync_copy(zero_hbm.at[nsl], agg_sh.at[nsl])
    plsc.subcore_barrier()

    def process(r0, nrows):
        pltpu.sync_copy(row_hbm.at[pl.ds(r0, nrows)],
                        rowv.at[pl.ds(0, nrows)])
        pltpu.sync_copy(col_hbm.at[pl.ds(r0, nrows)],
                        colv.at[pl.ds(0, nrows)])
        pltpu.sync_copy(ew_hbm.at[pl.ds(r0 * LANES, nrows * LANES)],
                        wv.at[pl.ds(0, nrows * LANES)])
        for j in range(nrows):
            pltpu.async_copy(y_hbm.at[rowv.at[j]],
                             xg.at[pl.ds(j * LANES, LANES)], sem).wait()

        def egrp(k, carry):
            lin = k * 16 + lax.iota(jnp.int32, 16)
            e = lin >> 3
            t = lin & 7
            s16 = plsc.load_gather(wv, [e])
            xv = plsc.load_gather(xg, [e, t])
            plsc.store_scatter(ctr, [e, t], s16 * xv)
            return carry

        lax.fori_loop(0, nrows * LANES * p // 16, egrp, 0)

        for j in range(nrows):
            pltpu.sync_copy(ctr.at[pl.ds(j * LANES, LANES)],
                            agg_sh.at[colv.at[j]], add=True)

    def chunk(ci, carry):
        process(wid * rpw + ci * ck, ck)
        return carry

    lax.fori_loop(0, nchunk, chunk, 0)

    @pl.when(wid < extra)
    def _():
        process(rpw * NW + wid, 1)

    plsc.subcore_barrier()
    pltpu.sync_copy(agg_sh.at[nsl], aggp_hbm.at[cid, nsl])


def _dinv_body(degp_ref, xt3_ref, dinv_ref, yt3_ref):
    s = degp_ref[0] + degp_ref[1]
    r0 = lax.rsqrt(s)
    di = r0 * (1.5 - 0.5 * s * r0 * r0)
    dinv_ref[...] = di
    xb = xt3_ref[...].astype(jnp.bfloat16).astype(jnp.float32)
    yt3_ref[...] = xb * di[None, :, :]


def _dense_body(aggp_ref, x_ref, dinv_ref, s_ref, g2_ref, wz_ref, bz_ref,
                wh_ref, bh_ref, lz_ref, lh_ref, lzb_ref, lhb_ref, pr_ref,
                lw_ref, lb_ref, out_ref):
    hi = lax.Precision.HIGHEST
    di = dinv_ref[...]
    xb = x_ref[...].astype(jnp.bfloat16).astype(jnp.float32)
    a = (aggp_ref[0] + aggp_ref[1] + di * xb) * di
    a256 = jnp.dot(a, s_ref[...], preferred_element_type=jnp.float32,
                   precision=hi)
    cz = (a256 * wz_ref[...] + bz_ref[...]).astype(jnp.bfloat16)
    ch_ = (a256 * wh_ref[...] + bh_ref[...]).astype(jnp.bfloat16)
    u = jnp.dot(cz, lz_ref[...], preferred_element_type=jnp.float32) + lzb_ref[...]
    v = jnp.dot(ch_, lh_ref[...], preferred_element_type=jnp.float32) + lhb_ref[...]
    w = (1.0 - jax.nn.sigmoid(u)) * jnp.tanh(v) * pr_ref[...]
    hid = w.shape[1] // 8
    acc = w[:, 0:hid]
    for t in range(1, 8):
        acc = acc + w[:, t * hid:(t + 1) * hid]
    h = jnp.maximum(acc, 0.0).astype(jnp.bfloat16)
    out_ref[...] = (jnp.dot(h, lw_ref[...], preferred_element_type=jnp.float32)
                    + lb_ref[...])


def kernel(x, edge_index, edge_weight, attention, Wz, bz, Wr, br, Wh, bh,
           Lzw, Lzb, Lrw, Lrb, Lhw, Lhb, lin_w, lin_b):
    n, p = x.shape
    e = edge_index.shape[1]
    hid = Wz.shape[1]
    assert e % LANES == 0
    erows = e // LANES
    rpw = erows // NW
    extra = erows - rpw * NW
    ck = 1
    for cand in range(15, 0, -1):
        if rpw % cand == 0:
            ck = cand
            break
    nchunk = rpw // ck
    align = NS * LANES
    npad = ((n + align - 1) // align) * align

    row2 = edge_index[0].reshape(erows, LANES)
    col2 = edge_index[1].reshape(erows, LANES)
    xp = jnp.concatenate([x, jnp.zeros((npad - n, p), jnp.float32)], axis=0)
    init_deg = jnp.concatenate(
        [jnp.ones((1, npad), jnp.float32), jnp.zeros((1, npad), jnp.float32)],
        axis=0)
    zero8 = jnp.zeros((npad, p), jnp.float32)

    mesh = plsc.VectorSubcoreMesh(core_axis_name="c", subcore_axis_name="s")
    sc_params = pltpu.CompilerParams(use_tc_tiling_on_sc=False,
                                     needs_layout_passes=False)

    deg_fn = pl.kernel(
        functools.partial(_deg_body, npad=npad, rpw=rpw, nchunk=nchunk,
                          ck=ck, extra=extra),
        out_type=jax.ShapeDtypeStruct((NC, npad), jnp.float32),
        mesh=mesh,
        scratch_types=[
            pltpu.VMEM_SHARED((npad,), jnp.float32),
            pltpu.VMEM((ck, LANES), jnp.int32),
            pltpu.VMEM((ck * LANES,), jnp.float32),
        ],
        compiler_params=sc_params,
    )
    degp = deg_fn(col2, edge_weight, init_deg)

    dinv2, yt3 = pl.pallas_call(
        _dinv_body,
        out_shape=(
            jax.ShapeDtypeStruct((npad // LANES, LANES), jnp.float32),
            jax.ShapeDtypeStruct((p, npad // LANES, LANES), jnp.float32),
        ),
    )(degp.reshape(NC, npad // LANES, LANES),
      xp.T.reshape(p, npad // LANES, LANES))
    y = yt3.reshape(p, npad).T

    agg_fn = pl.kernel(
        functools.partial(_agg_body, npad=npad, p=p, rpw=rpw, nchunk=nchunk,
                          ck=ck, extra=extra),
        out_type=jax.ShapeDtypeStruct((NC, npad, p), jnp.float32),
        mesh=mesh,
        scratch_types=[
            pltpu.VMEM_SHARED((npad, p), jnp.float32),
            pltpu.VMEM((ck, LANES), jnp.int32),
            pltpu.VMEM((ck, LANES), jnp.int32),
            pltpu.VMEM((ck * LANES,), jnp.float32),
            pltpu.VMEM((ck * LANES, p), jnp.float32),
            pltpu.VMEM((ck * LANES, p), jnp.float32),
            pltpu.SemaphoreType.DMA,
        ],
        compiler_params=sc_params,
    )
    aggp = agg_fn(row2, col2, edge_weight, y, zero8)

    probs = jax.nn.softmax(attention)
    bf = jnp.bfloat16
    wz256 = jnp.tile(Wz[0].astype(bf).astype(jnp.float32), p)[None, :]
    bz256 = jnp.tile(bz, p)[None, :]
    wh256 = jnp.tile(Wh[0].astype(bf).astype(jnp.float32), p)[None, :]
    bh256 = jnp.tile(bh, p)[None, :]
    eye_p = jnp.eye(p, dtype=jnp.float32)
    lzblk = jnp.kron(eye_p, Lzw[:hid]).astype(bf)
    lhblk = jnp.kron(eye_p, Lhw[:hid]).astype(bf)
    lzb256 = jnp.tile(Lzb, p)[None, :]
    lhb256 = jnp.tile(Lhb, p)[None, :]
    pr = jnp.repeat(probs, hid)[None, :]
    sel = jnp.kron(eye_p, jnp.ones((1, hid), jnp.float32))
    g2 = jnp.kron(jnp.ones((p, 1), jnp.float32), jnp.eye(hid, dtype=jnp.float32))
    ph = p * hid

    br_rows = npad // 16
    grid = 16
    out_pad = pl.pallas_call(
        _dense_body,
        grid=(grid,),
        in_specs=[
            pl.BlockSpec((NC, br_rows, p), lambda i: (0, i, 0)),
            pl.BlockSpec((br_rows, p), lambda i: (i, 0)),
            pl.BlockSpec((br_rows, 1), lambda i: (i, 0)),
            pl.BlockSpec((p, ph), lambda i: (0, 0)),
            pl.BlockSpec((ph, hid), lambda i: (0, 0)),
            pl.BlockSpec((1, ph), lambda i: (0, 0)),
            pl.BlockSpec((1, ph), lambda i: (0, 0)),
            pl.BlockSpec((1, ph), lambda i: (0, 0)),
            pl.BlockSpec((1, ph), lambda i: (0, 0)),
            pl.BlockSpec((ph, ph), lambda i: (0, 0)),
            pl.BlockSpec((ph, ph), lambda i: (0, 0)),
            pl.BlockSpec((1, ph), lambda i: (0, 0)),
            pl.BlockSpec((1, ph), lambda i: (0, 0)),
            pl.BlockSpec((1, ph), lambda i: (0, 0)),
            pl.BlockSpec((hid, 1), lambda i: (0, 0)),
            pl.BlockSpec((1, 1), lambda i: (0, 0)),
        ],
        out_specs=pl.BlockSpec((br_rows, 1), lambda i: (i, 0)),
        out_shape=jax.ShapeDtypeStruct((npad, 1), jnp.float32),
    )(aggp, xp, dinv2.reshape(npad, 1), sel, g2, wz256, bz256, wh256, bh256,
      lzblk, lhblk, lzb256, lhb256, pr, lin_w.astype(bf), lin_b.reshape(1, 1))

    return out_pad[:n]
